# Initial kernel scaffold; baseline (speedup 1.0000x reference)
import jax
import jax.numpy as jnp
from jax.experimental import pallas as pl


def kernel(img, lut):
    raise NotImplementedError("write your pallas kernel here")


# speedup vs baseline: 28.6728x; 28.6728x over previous
import functools

import jax
import jax.numpy as jnp
from jax import lax
from jax.experimental import pallas as pl
from jax.experimental.pallas import tpu as pltpu
from jax.experimental.pallas import tpu_sc as plsc

LUT_L = 17
L3, L2 = LUT_L ** 3, LUT_L
Q = 16
BORDER = 127.0
C, H, W = 3, 256, 256
NC, NS = 2, 16
NW = NC * NS
ROWS = C * H
ROWS_PER = ROWS // NW
OUT_ROW = 16 * W
NGROUP = W // 16
NIDX = 5 * W
GCHUNK = 128


def _pcode(s):
    m = {"a": 0, "b": 1, "c": 2, "d": 3}
    v = 0
    for ch in s:
        v = v * 4 + m[ch]
    return v


def _quant_body(l_ref, o_ref):
    x = l_ref[...] * BORDER
    o_ref[...] = jnp.clip(jnp.round(x), -BORDER, BORDER)


def _quantize(lut):
    flat = lut.reshape(-1)
    n = flat.shape[0]
    rows = (n + 1023) // 1024
    padded = jnp.pad(flat, (0, rows * 1024 - n)).reshape(rows, 1024)
    q = pl.pallas_call(
        _quant_body,
        out_shape=jax.ShapeDtypeStruct((rows, 1024), jnp.float32),
    )(padded)
    return q.reshape(-1)[:n].reshape(lut.shape)


_MESH = plsc.VectorSubcoreMesh(core_axis_name="c", subcore_axis_name="s")


def _sc_body(va_h, vb_h, vc_h, vd_h, lq_h, out_h,
             va_v, vb_v, vc_v, vd_v, idx_v, w_v, rows_v, orow_v,
             gsem, isem, osem):
    wid = lax.axis_index("s") * NC + lax.axis_index("c")

    lane = lax.iota(jnp.int32, 16)
    pattern = (lane >> 2) * (4 * W) + (lane & 3)

    def row_body(r, carry):
        row = wid * ROWS_PER + r
        base = row * W

        in_cps = [pltpu.make_async_copy(h.at[pl.ds(base, W)], v, isem)
                  for h, v in ((va_h, va_v), (vb_h, vb_v),
                               (vc_h, vc_v), (vd_h, vd_v))]
        for cp in in_cps:
            cp.start()
        for cp in in_cps:
            cp.wait()

        for g in range(NGROUP):
            sl = pl.ds(g * 16, 16)
            va = va_v[sl]
            vb = vb_v[sl]
            vc = vc_v[sl]
            vd = vd_v[sl]
            Aa, Fa = va >> 4, va & 15
            Ab, Fb = vb >> 4, vb & 15
            Ac, Fc = vc >> 4, vc & 15
            Ad, Fd = vd >> 4, vd & 15
            i0 = Aa * (LUT_L ** 3) + Ab * (LUT_L ** 2) + Ac * LUT_L + Ad
            db = (Aa + 1 - Ab) * (LUT_L ** 2)
            Lab = Fa > Fb
            Lac = Fa > Fc
            Lad = Fa > Fd
            Lbc = Fb > Fc
            Lbd = Fb > Fd
            Lcd = Fc > Fd
            w_ = jnp.where
            code1 = w_(Lcd, _pcode("abcd"),
                       w_(Lbd, _pcode("abdc"),
                          w_(Lad, _pcode("adbc"), _pcode("dabc"))))
            code2 = w_(Lbd, _pcode("acbd"),
                       w_(Lcd, _pcode("acdb"),
                          w_(Lad, _pcode("adcb"), _pcode("dacb"))))
            code3 = w_(Lad, _pcode("cabd"),
                       w_(Lcd, _pcode("cdab"), _pcode("dcab")))
            code4 = w_(Lcd, _pcode("bacd"),
                       w_(Lad, _pcode("badc"),
                          w_(Lbd, _pcode("bdac"), _pcode("dbac"))))
            code5 = w_(Lad, _pcode("bcad"),
                       w_(Lcd, _pcode("bcda"),
                          w_(Lbd, _pcode("bdca"), _pcode("dbca"))))
            code6 = w_(Lad, _pcode("cbad"),
                       w_(Lbd, _pcode("cbda"),
                          w_(Lcd, _pcode("cdba"), _pcode("dcba"))))
            code = w_(Lab,
                      w_(Lbc, code1, w_(Lac, code2, code3)),
                      w_(Lac, code4, w_(Lbc, code5, code6)))
            o1 = code >> 6
            o2 = (code >> 4) & 3
            o3 = (code >> 2) & 3
            o4 = code & 3

            def sel4(o, xa, xb, xc, xd):
                return w_(o == 0, xa, w_(o == 1, xb, w_(o == 2, xc, xd)))

            l1 = sel4(o1, Fa, Fb, Fc, Fd)
            l2 = sel4(o2, Fa, Fb, Fc, Fd)
            l3 = sel4(o3, Fa, Fb, Fc, Fd)
            l4 = sel4(o4, Fa, Fb, Fc, Fd)
            i1 = i0 + sel4(o1, LUT_L ** 3, db, LUT_L, 1)
            i2 = i1 + sel4(o2, LUT_L ** 3, db, LUT_L, 1)
            i3 = i2 + sel4(o3, LUT_L ** 3, db, LUT_L, 1)
            i4 = i3 + sel4(o4, LUT_L ** 3, db, LUT_L, 1)
            idx_v[pl.ds(0 * W + g * 16, 16)] = i0
            idx_v[pl.ds(1 * W + g * 16, 16)] = i1
            idx_v[pl.ds(2 * W + g * 16, 16)] = i2
            idx_v[pl.ds(3 * W + g * 16, 16)] = i3
            idx_v[pl.ds(4 * W + g * 16, 16)] = i4
            w_v[pl.ds(0 * W + g * 16, 16)] = (Q - l1).astype(jnp.float32)
            w_v[pl.ds(1 * W + g * 16, 16)] = (l1 - l2).astype(jnp.float32)
            w_v[pl.ds(2 * W + g * 16, 16)] = (l2 - l3).astype(jnp.float32)
            w_v[pl.ds(3 * W + g * 16, 16)] = (l3 - l4).astype(jnp.float32)
            w_v[pl.ds(4 * W + g * 16, 16)] = l4.astype(jnp.float32)

        g_cps = [pltpu.make_async_copy(
                     lq_h.at[idx_v.at[pl.ds(j * GCHUNK, GCHUNK)]],
                     rows_v.at[pl.ds(j * GCHUNK, GCHUNK)], gsem)
                 for j in range(NIDX // GCHUNK)]
        for cp in g_cps:
            cp.start()
        for cp in g_cps:
            cp.wait()

        def cgroup_body(g, c2):
            p0 = g * 16
            wvecs = [w_v[pl.ds(k * W + p0, 16)] for k in range(5)]
            bases = [(k * W + p0) + lane for k in range(5)]
            pxbase = (p0 + lane) * 4
            for j in range(16):
                jj = jnp.full((16,), j, jnp.int32)
                acc = wvecs[0] * plsc.load_gather(rows_v, [bases[0], jj])
                for k in range(1, 5):
                    acc = acc + wvecs[k] * plsc.load_gather(
                        rows_v, [bases[k], jj])
                oidx = pxbase + ((j >> 2) * (4 * W) + (j & 3))
                plsc.store_scatter(orow_v, [oidx], acc)
            return c2

        lax.fori_loop(0, NGROUP, cgroup_body, 0)

        ocp = pltpu.make_async_copy(
            orow_v, out_h.at[pl.ds(row * OUT_ROW, OUT_ROW)], osem)
        ocp.start()
        ocp.wait()
        return carry

    lax.fori_loop(0, ROWS_PER, row_body, 0)


_sc_kernel = functools.partial(
    pl.kernel,
    out_type=jax.ShapeDtypeStruct((C * H * OUT_ROW,), jnp.float32),
    mesh=_MESH,
    compiler_params=pltpu.CompilerParams(
        needs_layout_passes=False, use_tc_tiling_on_sc=False),
    scratch_types=[
        pltpu.VMEM((W,), jnp.int32),
        pltpu.VMEM((W,), jnp.int32),
        pltpu.VMEM((W,), jnp.int32),
        pltpu.VMEM((W,), jnp.int32),
        pltpu.VMEM((NIDX,), jnp.int32),
        pltpu.VMEM((5 * W,), jnp.float32),
        pltpu.VMEM((NIDX, 16), jnp.float32),
        pltpu.VMEM((OUT_ROW,), jnp.float32),
        pltpu.SemaphoreType.DMA,
        pltpu.SemaphoreType.DMA,
        pltpu.SemaphoreType.DMA,
    ],
)(_sc_body)


def kernel(img, lut):
    v = img.astype(jnp.int32)
    va = v[:, :, 0:H, 0:W].reshape(-1)
    vb = v[:, :, 0:H, 1:1 + W].reshape(-1)
    vc = v[:, :, 1:1 + H, 0:W].reshape(-1)
    vd = v[:, :, 1:1 + H, 1:1 + W].reshape(-1)
    lq = _quantize(lut)
    out = _sc_kernel(va, vb, vc, vd, lq)
    return out.reshape(1, C, 4 * H, 4 * W)


# speedup vs baseline: 43.3364x; 1.5114x over previous
import functools

import jax
import jax.numpy as jnp
from jax import lax
from jax.experimental import pallas as pl
from jax.experimental.pallas import tpu as pltpu
from jax.experimental.pallas import tpu_sc as plsc

LUT_L = 17
Q = 16
BORDER = 127.0
C, H, W = 3, 256, 256
NC, NS = 2, 16
NW = NC * NS
ROWS = C * H
ROWS_PER = ROWS // NW
OUT_ROW = 16 * W
NGROUP = W // 16
NIDX = 5 * W
GCHUNK = 128
IMG_STRIDE = 264
QROWS = 10448
TAB_ROWS = QROWS * 128 // 16


def _pcode(s):
    m = {"a": 0, "b": 1, "c": 2, "d": 3}
    v = 0
    for ch in s:
        v = v * 4 + m[ch]
    return v


def _quant_body(l_ref, o_ref):
    x = l_ref[...] * BORDER
    o_ref[...] = jnp.clip(jnp.round(x), -BORDER, BORDER)


def _quantize(lut):
    flat = lut.reshape(-1)
    padded = jnp.pad(flat, (0, QROWS * 128 - flat.shape[0]))
    q = pl.pallas_call(
        _quant_body,
        out_shape=jax.ShapeDtypeStruct((QROWS, 128), jnp.float32),
    )(padded.reshape(QROWS, 128))
    return q.reshape(TAB_ROWS, 16)


_MESH = plsc.VectorSubcoreMesh(core_axis_name="c", subcore_axis_name="s")

_LANE = None


def _phase_a(rowa_v, rowb_v, idx_v, w_v, lane):
    for g in range(NGROUP):
        sl = pl.ds(g * 16, 16)
        va = rowa_v[sl]
        vb = plsc.load_gather(rowa_v, [lane + (g * 16 + 1)])
        vc = rowb_v[sl]
        vd = plsc.load_gather(rowb_v, [lane + (g * 16 + 1)])
        Aa, Fa = va >> 4, va & 15
        Ab, Fb = vb >> 4, vb & 15
        Ac, Fc = vc >> 4, vc & 15
        Ad, Fd = vd >> 4, vd & 15
        i0 = Aa * (LUT_L ** 3) + Ab * (LUT_L ** 2) + Ac * LUT_L + Ad
        db = (Aa + 1 - Ab) * (LUT_L ** 2)
        Lab = Fa > Fb
        Lac = Fa > Fc
        Lad = Fa > Fd
        Lbc = Fb > Fc
        Lbd = Fb > Fd
        Lcd = Fc > Fd
        w_ = jnp.where
        code1 = w_(Lcd, _pcode("abcd"),
                   w_(Lbd, _pcode("abdc"),
                      w_(Lad, _pcode("adbc"), _pcode("dabc"))))
        code2 = w_(Lbd, _pcode("acbd"),
                   w_(Lcd, _pcode("acdb"),
                      w_(Lad, _pcode("adcb"), _pcode("dacb"))))
        code3 = w_(Lad, _pcode("cabd"),
                   w_(Lcd, _pcode("cdab"), _pcode("dcab")))
        code4 = w_(Lcd, _pcode("bacd"),
                   w_(Lad, _pcode("badc"),
                      w_(Lbd, _pcode("bdac"), _pcode("dbac"))))
        code5 = w_(Lad, _pcode("bcad"),
                   w_(Lcd, _pcode("bcda"),
                      w_(Lbd, _pcode("bdca"), _pcode("dbca"))))
        code6 = w_(Lad, _pcode("cbad"),
                   w_(Lbd, _pcode("cbda"),
                      w_(Lcd, _pcode("cdba"), _pcode("dcba"))))
        code = w_(Lab,
                  w_(Lbc, code1, w_(Lac, code2, code3)),
                  w_(Lac, code4, w_(Lbc, code5, code6)))
        o1 = code >> 6
        o2 = (code >> 4) & 3
        o3 = (code >> 2) & 3
        o4 = code & 3

        def sel4(o, xa, xb, xc, xd):
            return w_(o == 0, xa, w_(o == 1, xb, w_(o == 2, xc, xd)))

        l1 = sel4(o1, Fa, Fb, Fc, Fd)
        l2 = sel4(o2, Fa, Fb, Fc, Fd)
        l3 = sel4(o3, Fa, Fb, Fc, Fd)
        l4 = sel4(o4, Fa, Fb, Fc, Fd)
        i1 = i0 + sel4(o1, LUT_L ** 3, db, LUT_L, 1)
        i2 = i1 + sel4(o2, LUT_L ** 3, db, LUT_L, 1)
        i3 = i2 + sel4(o3, LUT_L ** 3, db, LUT_L, 1)
        i4 = i3 + sel4(o4, LUT_L ** 3, db, LUT_L, 1)
        idx_v[pl.ds(0 * W + g * 16, 16)] = i0
        idx_v[pl.ds(1 * W + g * 16, 16)] = i1
        idx_v[pl.ds(2 * W + g * 16, 16)] = i2
        idx_v[pl.ds(3 * W + g * 16, 16)] = i3
        idx_v[pl.ds(4 * W + g * 16, 16)] = i4
        w_v[pl.ds(0 * W + g * 16, 16)] = (Q - l1).astype(jnp.float32)
        w_v[pl.ds(1 * W + g * 16, 16)] = (l1 - l2).astype(jnp.float32)
        w_v[pl.ds(2 * W + g * 16, 16)] = (l2 - l3).astype(jnp.float32)
        w_v[pl.ds(3 * W + g * 16, 16)] = (l3 - l4).astype(jnp.float32)
        w_v[pl.ds(4 * W + g * 16, 16)] = l4.astype(jnp.float32)


def _phase_c(idx_unused, w_v, rows_v, orow_v, lane):
    def cgroup_body(g, c2):
        p0 = g * 16
        wvecs = [w_v[pl.ds(k * W + p0, 16)] for k in range(5)]
        bases = [(k * W + p0) + lane for k in range(5)]
        pxbase = (p0 + lane) * 4
        for j in range(16):
            jj = jnp.full((16,), j, jnp.int32)
            acc = wvecs[0] * plsc.load_gather(rows_v, [bases[0], jj])
            for k in range(1, 5):
                acc = acc + wvecs[k] * plsc.load_gather(rows_v, [bases[k], jj])
            oidx = pxbase + ((j >> 2) * (4 * W) + (j & 3))
            plsc.store_scatter(orow_v, [oidx], acc)
        return c2

    lax.fori_loop(0, NGROUP, cgroup_body, 0)


def _sc_body(img_h, lq_h, out_h,
             ra0, rb0, ra1, rb1, idx0, idx1, w0, w1,
             rows0, rows1, orow0, orow1,
             isem0, isem1, gsem0, gsem1, osem0, osem1):
    wid = lax.axis_index("s") * NC + lax.axis_index("c")
    row_base = wid * ROWS_PER
    lane = lax.iota(jnp.int32, 16)

    def in_copies(row, ra, rb, sem):
        c = row >> 8
        h = row & 255
        off = (c * 257 + h) * IMG_STRIDE
        return (pltpu.make_async_copy(
                    img_h.at[pl.ds(off, IMG_STRIDE)], ra, sem),
                pltpu.make_async_copy(
                    img_h.at[pl.ds(off + IMG_STRIDE, IMG_STRIDE)], rb, sem))

    def fire_in(row, ra, rb, sem):
        for cp in in_copies(row, ra, rb, sem):
            cp.start()

    def wait_in(row, ra, rb, sem):
        for cp in in_copies(row, ra, rb, sem):
            cp.wait()

    def g_copies(idx_v, rows_v, sem):
        return [pltpu.make_async_copy(
                    lq_h.at[idx_v.at[pl.ds(j * GCHUNK, GCHUNK)]],
                    rows_v.at[pl.ds(j * GCHUNK, GCHUNK)], sem)
                for j in range(NIDX // GCHUNK)]

    def fire_g(idx_v, rows_v, sem):
        for cp in g_copies(idx_v, rows_v, sem):
            cp.start()

    def wait_g(idx_v, rows_v, sem):
        for cp in g_copies(idx_v, rows_v, sem):
            cp.wait()

    def out_copy(row, orow_v, sem):
        return pltpu.make_async_copy(
            orow_v, out_h.at[pl.ds(row * OUT_ROW, OUT_ROW)], sem)

    fire_in(row_base, ra0, rb0, isem0)
    wait_in(row_base, ra0, rb0, isem0)
    _phase_a(ra0, rb0, idx0, w0, lane)
    fire_g(idx0, rows0, gsem0)
    fire_in(row_base + 1, ra1, rb1, isem1)

    NB = ROWS_PER // 2

    def body(i, carry):
        r0 = row_base + 2 * i
        r1 = r0 + 1
        wait_in(r1, ra1, rb1, isem1)
        _phase_a(ra1, rb1, idx1, w1, lane)
        fire_g(idx1, rows1, gsem1)
        @pl.when(i < NB - 1)
        def _():
            fire_in(r0 + 2, ra0, rb0, isem0)
        @pl.when(i > 0)
        def _():
            out_copy(r0, orow0, osem0).wait()
        wait_g(idx0, rows0, gsem0)
        _phase_c(idx0, w0, rows0, orow0, lane)
        out_copy(r0, orow0, osem0).start()
        @pl.when(i < NB - 1)
        def _():
            wait_in(r0 + 2, ra0, rb0, isem0)
            _phase_a(ra0, rb0, idx0, w0, lane)
            fire_g(idx0, rows0, gsem0)
            fire_in(r1 + 2, ra1, rb1, isem1)
        @pl.when(i > 0)
        def _():
            out_copy(r1, orow1, osem1).wait()
        wait_g(idx1, rows1, gsem1)
        _phase_c(idx1, w1, rows1, orow1, lane)
        out_copy(r1, orow1, osem1).start()
        return carry

    lax.fori_loop(0, NB, body, 0)

    out_copy(row_base + ROWS_PER - 2, orow0, osem0).wait()
    out_copy(row_base + ROWS_PER - 1, orow1, osem1).wait()


_sc_kernel = functools.partial(
    pl.kernel,
    out_type=jax.ShapeDtypeStruct((C * H * OUT_ROW,), jnp.float32),
    mesh=_MESH,
    compiler_params=pltpu.CompilerParams(
        needs_layout_passes=False, use_tc_tiling_on_sc=False),
    scratch_types=[
        pltpu.VMEM((IMG_STRIDE,), jnp.int32),
        pltpu.VMEM((IMG_STRIDE,), jnp.int32),
        pltpu.VMEM((IMG_STRIDE,), jnp.int32),
        pltpu.VMEM((IMG_STRIDE,), jnp.int32),
        pltpu.VMEM((NIDX,), jnp.int32),
        pltpu.VMEM((NIDX,), jnp.int32),
        pltpu.VMEM((NIDX,), jnp.float32),
        pltpu.VMEM((NIDX,), jnp.float32),
        pltpu.VMEM((NIDX, 16), jnp.float32),
        pltpu.VMEM((NIDX, 16), jnp.float32),
        pltpu.VMEM((OUT_ROW,), jnp.float32),
        pltpu.VMEM((OUT_ROW,), jnp.float32),
        pltpu.SemaphoreType.DMA,
        pltpu.SemaphoreType.DMA,
        pltpu.SemaphoreType.DMA,
        pltpu.SemaphoreType.DMA,
        pltpu.SemaphoreType.DMA,
        pltpu.SemaphoreType.DMA,
    ],
)(_sc_body)


def kernel(img, lut):
    imgp = jnp.pad(img[0].astype(jnp.int32),
                   ((0, 0), (0, 0), (0, IMG_STRIDE - 257))).reshape(-1)
    lq = _quantize(lut)
    out = _sc_kernel(imgp, lq)
    return out.reshape(1, C, 4 * H, 4 * W)


# speedup vs baseline: 71.6522x; 1.6534x over previous
import functools

import jax
import jax.numpy as jnp
from jax import lax
from jax.experimental import pallas as pl
from jax.experimental.pallas import tpu as pltpu
from jax.experimental.pallas import tpu_sc as plsc

LUT_L = 17
Q = 16
BORDER = 127.0
C, H, W = 3, 256, 256
NC, NS = 2, 16
NW = NC * NS
ROWS = C * H
ROWS_PER = ROWS // NW
OUT_ROW = 16 * W
NGROUP = W // 16
NIDX = 5 * W
GCHUNK = 128
IMG_STRIDE = 264
QROWS = 10448
TAB_ROWS = QROWS * 128 // 16


def _pcode(s):
    m = {"a": 0, "b": 1, "c": 2, "d": 3}
    v = 0
    for ch in s:
        v = v * 4 + m[ch]
    return v


def _quant_body(l_ref, o_ref):
    x = l_ref[...] * BORDER
    o_ref[...] = jnp.clip(jnp.round(x), -BORDER, BORDER)


def _quantize(lut):
    flat = lut.reshape(-1)
    padded = jnp.pad(flat, (0, QROWS * 128 - flat.shape[0]))
    q = pl.pallas_call(
        _quant_body,
        out_shape=jax.ShapeDtypeStruct((QROWS, 128), jnp.float32),
    )(padded.reshape(QROWS, 128))
    return q.reshape(TAB_ROWS, 16)


_MESH = plsc.VectorSubcoreMesh(core_axis_name="c", subcore_axis_name="s")

_LANE = None


def _phase_a(rowa_v, rowb_v, idx_v, w_v, lane):
    for g in range(NGROUP):
        sl = pl.ds(g * 16, 16)
        va = rowa_v[sl]
        vb = plsc.load_gather(rowa_v, [lane + (g * 16 + 1)])
        vc = rowb_v[sl]
        vd = plsc.load_gather(rowb_v, [lane + (g * 16 + 1)])
        Aa, Fa = va >> 4, va & 15
        Ab, Fb = vb >> 4, vb & 15
        Ac, Fc = vc >> 4, vc & 15
        Ad, Fd = vd >> 4, vd & 15
        i0 = Aa * (LUT_L ** 3) + Ab * (LUT_L ** 2) + Ac * LUT_L + Ad
        db = (Aa + 1 - Ab) * (LUT_L ** 2)
        Lab = Fa > Fb
        Lac = Fa > Fc
        Lad = Fa > Fd
        Lbc = Fb > Fc
        Lbd = Fb > Fd
        Lcd = Fc > Fd
        w_ = jnp.where
        code1 = w_(Lcd, _pcode("abcd"),
                   w_(Lbd, _pcode("abdc"),
                      w_(Lad, _pcode("adbc"), _pcode("dabc"))))
        code2 = w_(Lbd, _pcode("acbd"),
                   w_(Lcd, _pcode("acdb"),
                      w_(Lad, _pcode("adcb"), _pcode("dacb"))))
        code3 = w_(Lad, _pcode("cabd"),
                   w_(Lcd, _pcode("cdab"), _pcode("dcab")))
        code4 = w_(Lcd, _pcode("bacd"),
                   w_(Lad, _pcode("badc"),
                      w_(Lbd, _pcode("bdac"), _pcode("dbac"))))
        code5 = w_(Lad, _pcode("bcad"),
                   w_(Lcd, _pcode("bcda"),
                      w_(Lbd, _pcode("bdca"), _pcode("dbca"))))
        code6 = w_(Lad, _pcode("cbad"),
                   w_(Lbd, _pcode("cbda"),
                      w_(Lcd, _pcode("cdba"), _pcode("dcba"))))
        code = w_(Lab,
                  w_(Lbc, code1, w_(Lac, code2, code3)),
                  w_(Lac, code4, w_(Lbc, code5, code6)))
        o1 = code >> 6
        o2 = (code >> 4) & 3
        o3 = (code >> 2) & 3
        o4 = code & 3

        def sel4(o, xa, xb, xc, xd):
            return w_(o == 0, xa, w_(o == 1, xb, w_(o == 2, xc, xd)))

        l1 = sel4(o1, Fa, Fb, Fc, Fd)
        l2 = sel4(o2, Fa, Fb, Fc, Fd)
        l3 = sel4(o3, Fa, Fb, Fc, Fd)
        l4 = sel4(o4, Fa, Fb, Fc, Fd)
        i1 = i0 + sel4(o1, LUT_L ** 3, db, LUT_L, 1)
        i2 = i1 + sel4(o2, LUT_L ** 3, db, LUT_L, 1)
        i3 = i2 + sel4(o3, LUT_L ** 3, db, LUT_L, 1)
        i4 = i3 + sel4(o4, LUT_L ** 3, db, LUT_L, 1)
        idx_v[pl.ds(0 * W + g * 16, 16)] = i0
        idx_v[pl.ds(1 * W + g * 16, 16)] = i1
        idx_v[pl.ds(2 * W + g * 16, 16)] = i2
        idx_v[pl.ds(3 * W + g * 16, 16)] = i3
        idx_v[pl.ds(4 * W + g * 16, 16)] = i4
        w_v[pl.ds(0 * W + g * 16, 16)] = (Q - l1).astype(jnp.float32)
        w_v[pl.ds(1 * W + g * 16, 16)] = (l1 - l2).astype(jnp.float32)
        w_v[pl.ds(2 * W + g * 16, 16)] = (l2 - l3).astype(jnp.float32)
        w_v[pl.ds(3 * W + g * 16, 16)] = (l3 - l4).astype(jnp.float32)
        w_v[pl.ds(4 * W + g * 16, 16)] = l4.astype(jnp.float32)


def _phase_c(idx_unused, w_v, rows_v, orow_v, lane):
    pattern = (lane >> 2) * (4 * W) + (lane & 3)
    zeros = jnp.zeros((16,), jnp.int32)

    def px_body(p, c2):
        sp = zeros + p
        acc = (plsc.load_gather(w_v, [sp])
               * plsc.load_gather(rows_v, [sp, lane]))
        for k in range(1, 5):
            acc = acc + (plsc.load_gather(w_v, [sp + k * W])
                         * plsc.load_gather(rows_v, [sp + k * W, lane]))
        plsc.store_scatter(orow_v, [pattern + p * 4], acc)
        return c2

    lax.fori_loop(0, W, px_body, 0)


def _sc_body(img_h, lq_h, out_h,
             ra0, rb0, ra1, rb1, idx0, idx1, w0, w1,
             rows0, rows1, orow0, orow1,
             isem0, isem1, gsem0, gsem1, osem0, osem1):
    wid = lax.axis_index("s") * NC + lax.axis_index("c")
    row_base = wid * ROWS_PER
    lane = lax.iota(jnp.int32, 16)

    def in_copies(row, ra, rb, sem):
        c = row >> 8
        h = row & 255
        off = (c * 257 + h) * IMG_STRIDE
        return (pltpu.make_async_copy(
                    img_h.at[pl.ds(off, IMG_STRIDE)], ra, sem),
                pltpu.make_async_copy(
                    img_h.at[pl.ds(off + IMG_STRIDE, IMG_STRIDE)], rb, sem))

    def fire_in(row, ra, rb, sem):
        for cp in in_copies(row, ra, rb, sem):
            cp.start()

    def wait_in(row, ra, rb, sem):
        for cp in in_copies(row, ra, rb, sem):
            cp.wait()

    def g_copies(idx_v, rows_v, sem):
        return [pltpu.make_async_copy(
                    lq_h.at[idx_v.at[pl.ds(j * GCHUNK, GCHUNK)]],
                    rows_v.at[pl.ds(j * GCHUNK, GCHUNK)], sem)
                for j in range(NIDX // GCHUNK)]

    def fire_g(idx_v, rows_v, sem):
        for cp in g_copies(idx_v, rows_v, sem):
            cp.start()

    def wait_g(idx_v, rows_v, sem):
        for cp in g_copies(idx_v, rows_v, sem):
            cp.wait()

    def out_copy(row, orow_v, sem):
        return pltpu.make_async_copy(
            orow_v, out_h.at[pl.ds(row * OUT_ROW, OUT_ROW)], sem)

    fire_in(row_base, ra0, rb0, isem0)
    wait_in(row_base, ra0, rb0, isem0)
    _phase_a(ra0, rb0, idx0, w0, lane)
    fire_g(idx0, rows0, gsem0)
    fire_in(row_base + 1, ra1, rb1, isem1)

    NB = ROWS_PER // 2

    def body(i, carry):
        r0 = row_base + 2 * i
        r1 = r0 + 1
        wait_in(r1, ra1, rb1, isem1)
        _phase_a(ra1, rb1, idx1, w1, lane)
        fire_g(idx1, rows1, gsem1)
        @pl.when(i < NB - 1)
        def _():
            fire_in(r0 + 2, ra0, rb0, isem0)
        @pl.when(i > 0)
        def _():
            out_copy(r0, orow0, osem0).wait()
        wait_g(idx0, rows0, gsem0)
        _phase_c(idx0, w0, rows0, orow0, lane)
        out_copy(r0, orow0, osem0).start()
        @pl.when(i < NB - 1)
        def _():
            wait_in(r0 + 2, ra0, rb0, isem0)
            _phase_a(ra0, rb0, idx0, w0, lane)
            fire_g(idx0, rows0, gsem0)
            fire_in(r1 + 2, ra1, rb1, isem1)
        @pl.when(i > 0)
        def _():
            out_copy(r1, orow1, osem1).wait()
        wait_g(idx1, rows1, gsem1)
        _phase_c(idx1, w1, rows1, orow1, lane)
        out_copy(r1, orow1, osem1).start()
        return carry

    lax.fori_loop(0, NB, body, 0)

    out_copy(row_base + ROWS_PER - 2, orow0, osem0).wait()
    out_copy(row_base + ROWS_PER - 1, orow1, osem1).wait()


_sc_kernel = functools.partial(
    pl.kernel,
    out_type=jax.ShapeDtypeStruct((C * H * OUT_ROW,), jnp.float32),
    mesh=_MESH,
    compiler_params=pltpu.CompilerParams(
        needs_layout_passes=False, use_tc_tiling_on_sc=False),
    scratch_types=[
        pltpu.VMEM((IMG_STRIDE,), jnp.int32),
        pltpu.VMEM((IMG_STRIDE,), jnp.int32),
        pltpu.VMEM((IMG_STRIDE,), jnp.int32),
        pltpu.VMEM((IMG_STRIDE,), jnp.int32),
        pltpu.VMEM((NIDX,), jnp.int32),
        pltpu.VMEM((NIDX,), jnp.int32),
        pltpu.VMEM((NIDX,), jnp.float32),
        pltpu.VMEM((NIDX,), jnp.float32),
        pltpu.VMEM((NIDX, 16), jnp.float32),
        pltpu.VMEM((NIDX, 16), jnp.float32),
        pltpu.VMEM((OUT_ROW,), jnp.float32),
        pltpu.VMEM((OUT_ROW,), jnp.float32),
        pltpu.SemaphoreType.DMA,
        pltpu.SemaphoreType.DMA,
        pltpu.SemaphoreType.DMA,
        pltpu.SemaphoreType.DMA,
        pltpu.SemaphoreType.DMA,
        pltpu.SemaphoreType.DMA,
    ],
)(_sc_body)


def kernel(img, lut):
    imgp = jnp.pad(img[0].astype(jnp.int32),
                   ((0, 0), (0, 0), (0, IMG_STRIDE - 257))).reshape(-1)
    lq = _quantize(lut)
    out = _sc_kernel(imgp, lq)
    return out.reshape(1, C, 4 * H, 4 * W)
